# Initial kernel scaffold; baseline (speedup 1.0000x reference)
import jax
import jax.numpy as jnp
from jax.experimental import pallas as pl


def kernel(x, edge_index, W0, att_src0, att_dst0, b0, W1, att_src1, att_dst1, b1):
    raise NotImplementedError("write your pallas kernel here")


# speedup vs baseline: 17.4126x; 17.4126x over previous
import functools

import jax
import jax.numpy as jnp
from jax import lax
from jax.experimental import pallas as pl
from jax.experimental.pallas import tpu as pltpu
from jax.experimental.pallas import tpu_sc as plsc

N = 10000
E = 320000
D_IN = 128
HID = 128
HEADS = 8

NP = 10240
BN = 1280
NB = NP // BN
NC = 2
NS = 16
L = 16
BLK = 128
NBLKS = E // BLK
ROWS_PER_TILE = NP // NS


def _tc0_body(x_ref, w0_ref, asrc_ref, adst_ref, h0_ref, asT_ref, adT_ref):
    h = pl.program_id(0)
    hb = jnp.dot(x_ref[...], w0_ref[...], preferred_element_type=jnp.float32)
    h0_ref[0] = hb
    sel = lax.broadcasted_iota(jnp.int32, (HEADS, 1), 0) == h
    arow_s = jnp.sum(jnp.where(sel, asrc_ref[...], 0.0), axis=0, keepdims=True)
    arow_d = jnp.sum(jnp.where(sel, adst_ref[...], 0.0), axis=0, keepdims=True)
    asT_ref[0, 0] = jnp.sum(hb * arow_s, axis=1)
    adT_ref[0, 0] = jnp.sum(hb * arow_d, axis=1)


def _tc0(xp, W0, att_src0, att_dst0):
    return pl.pallas_call(
        _tc0_body,
        grid=(HEADS, NB),
        in_specs=[
            pl.BlockSpec((BN, D_IN), lambda h, nb: (nb, 0)),
            pl.BlockSpec((D_IN, HID), lambda h, nb: (0, h)),
            pl.BlockSpec((HEADS, HID), lambda h, nb: (0, 0)),
            pl.BlockSpec((HEADS, HID), lambda h, nb: (0, 0)),
        ],
        out_specs=[
            pl.BlockSpec((1, BN, HID), lambda h, nb: (h, nb, 0)),
            pl.BlockSpec((1, 1, BN), lambda h, nb: (h, 0, nb)),
            pl.BlockSpec((1, 1, BN), lambda h, nb: (h, 0, nb)),
        ],
        out_shape=[
            jax.ShapeDtypeStruct((HEADS, NP, HID), jnp.float32),
            jax.ShapeDtypeStruct((HEADS, 1, NP), jnp.float32),
            jax.ShapeDtypeStruct((HEADS, 1, NP), jnp.float32),
        ],
    )(xp, W0, att_src0, att_dst0)


def _zero_rows(rows):
    zv = jnp.zeros((L,), jnp.float32)

    def body(r, c):
        for j in range(HID // L):
            rows[r, pl.ds(j * L, L)] = zv
        return c

    lax.fori_loop(0, BLK, body, 0)


def _zero_tab(tab):
    zv = jnp.zeros((L,), jnp.float32)

    def body(i, c):
        tab[pl.ds(i * L, L)] = zv
        return c

    lax.fori_loop(0, NP // L, body, 0)


def _edge_block(src_hbm, dst_hbm, feat_hbm, asrc_tab, adst_tab, den_tab,
                rows, srcb, dstb, gidx, wb, boff, row_off):
    pltpu.sync_copy(src_hbm.at[pl.ds(boff, BLK)], srcb)
    pltpu.sync_copy(dst_hbm.at[pl.ds(boff, BLK)], dstb)
    for k in range(BLK // L):
        sl = pl.ds(k * L, L)
        s16 = srcb[sl]
        d16 = dstb[sl]
        a1 = plsc.load_gather(asrc_tab, [s16])
        a2 = plsc.load_gather(adst_tab, [d16])
        al = a1 + a2
        al = jnp.where(al >= 0.0, al, 0.2 * al)
        w16 = jnp.exp(al)
        wb[sl] = w16
        gidx[sl] = s16 + row_off
        plsc.addupdate_scatter(den_tab, [d16], w16)
    pltpu.sync_copy(feat_hbm.at[gidx], rows)

    def sbody(k, c):
        w16 = wb[pl.ds(k * L, L)]
        for ll in range(L):
            r = k * L + ll
            wl = w16[ll]
            for j in range(HID // L):
                sl2 = pl.ds(j * L, L)
                rows[r, sl2] = rows[r, sl2] * wl
        return c

    lax.fori_loop(0, BLK // L, sbody, 0)


def _sc0_body(src_hbm, dst_hbm, h0_hbm, asT_hbm, adT_hbm,
              num_hbm, den_hbm,
              accum, asrc_tab, adst_tab, den_tab, rows, srcb, dstb, gidx, wb):
    cid = lax.axis_index("c")
    sid = lax.axis_index("s")
    nblk = 156 + jnp.where(sid < NBLKS % NS, 1, 0).astype(jnp.int32)

    for hp in range(HEADS // NC):
        h = (2 * hp + cid).astype(jnp.int32)
        _zero_rows(rows)
        for q in range(ROWS_PER_TILE // BLK):
            pltpu.sync_copy(rows, accum.at[pl.ds(sid * ROWS_PER_TILE + q * BLK, BLK)])
        _zero_tab(den_tab)
        pltpu.sync_copy(asT_hbm.at[h], asrc_tab)
        pltpu.sync_copy(adT_hbm.at[h], adst_tab)
        plsc.subcore_barrier()

        row_off = h * NP

        def eb(i, c):
            boff = (sid + NS * i) * BLK
            _edge_block(src_hbm, dst_hbm, h0_hbm, asrc_tab, adst_tab, den_tab,
                        rows, srcb, dstb, gidx, wb, boff, row_off)
            pltpu.sync_copy(rows, accum.at[dstb], add=True)
            return c

        lax.fori_loop(0, nblk, eb, 0)
        plsc.subcore_barrier()
        rsl = pl.ds(sid * ROWS_PER_TILE, ROWS_PER_TILE)
        pltpu.sync_copy(accum.at[rsl], num_hbm.at[h, rsl])
        pltpu.sync_copy(den_tab, den_hbm.at[h, sid])
        plsc.subcore_barrier()


def _sc0(src, dst, h0flat, asT, adT):
    mesh = plsc.VectorSubcoreMesh(core_axis_name="c", subcore_axis_name="s",
                                  num_cores=NC, num_subcores=NS)
    f = pl.kernel(
        _sc0_body,
        out_type=[
            jax.ShapeDtypeStruct((HEADS, NP, HID), jnp.float32),
            jax.ShapeDtypeStruct((HEADS, NS, NP), jnp.float32),
        ],
        mesh=mesh,
        compiler_params=pltpu.CompilerParams(needs_layout_passes=False),
        scratch_types=[
            pltpu.VMEM_SHARED((NP, HID), jnp.float32),
            pltpu.VMEM((NP,), jnp.float32),
            pltpu.VMEM((NP,), jnp.float32),
            pltpu.VMEM((NP,), jnp.float32),
            pltpu.VMEM((BLK, HID), jnp.float32),
            pltpu.VMEM((BLK,), jnp.int32),
            pltpu.VMEM((BLK,), jnp.int32),
            pltpu.VMEM((BLK,), jnp.int32),
            pltpu.VMEM((BLK,), jnp.float32),
        ],
    )
    return f(src, dst, h0flat, asT, adT)


def _tcmid_body(num_ref, den_ref, b0_ref, w1_ref, a1s_ref, a1d_ref,
                h1_ref, asT_ref, adT_ref):
    den = jnp.sum(den_ref[...], axis=1)
    acc = jnp.zeros((BN, HID), jnp.float32)
    for h in range(HEADS):
        v = num_ref[h] / (den[h][:, None] + 1e-16) + b0_ref[h][None, :]
        v = jnp.where(v > 0.0, v, jnp.exp(v) - 1.0)
        acc = acc + jnp.dot(v, w1_ref[h], preferred_element_type=jnp.float32)
    h1_ref[...] = acc
    asT_ref[0] = jnp.sum(acc * a1s_ref[...], axis=1)
    adT_ref[0] = jnp.sum(acc * a1d_ref[...], axis=1)


def _tcmid(num0, den0, b0r, W1r, att_src1, att_dst1):
    return pl.pallas_call(
        _tcmid_body,
        grid=(NB,),
        in_specs=[
            pl.BlockSpec((HEADS, BN, HID), lambda nb: (0, nb, 0)),
            pl.BlockSpec((HEADS, NS, BN), lambda nb: (0, 0, nb)),
            pl.BlockSpec((HEADS, HID), lambda nb: (0, 0)),
            pl.BlockSpec((HEADS, HID, HID), lambda nb: (0, 0, 0)),
            pl.BlockSpec((1, HID), lambda nb: (0, 0)),
            pl.BlockSpec((1, HID), lambda nb: (0, 0)),
        ],
        out_specs=[
            pl.BlockSpec((BN, HID), lambda nb: (nb, 0)),
            pl.BlockSpec((1, BN), lambda nb: (0, nb)),
            pl.BlockSpec((1, BN), lambda nb: (0, nb)),
        ],
        out_shape=[
            jax.ShapeDtypeStruct((NP, HID), jnp.float32),
            jax.ShapeDtypeStruct((1, NP), jnp.float32),
            jax.ShapeDtypeStruct((1, NP), jnp.float32),
        ],
    )(num0, den0, b0r, W1r, att_src1, att_dst1)


def _sc1_body(src_hbm, dst_hbm, h1_hbm, asT_hbm, adT_hbm,
              num_hbm, den_hbm,
              accum, asrc_tab, adst_tab, den_tab, rows, srcb, dstb, gidx, wb):
    cid = lax.axis_index("c")
    sid = lax.axis_index("s")
    wid = sid * NC + cid
    nblk = 78 + jnp.where(wid < NBLKS % (NS * NC), 1, 0).astype(jnp.int32)

    _zero_rows(rows)
    for q in range(ROWS_PER_TILE // BLK):
        pltpu.sync_copy(rows, accum.at[pl.ds(sid * ROWS_PER_TILE + q * BLK, BLK)])
    _zero_tab(den_tab)
    pltpu.sync_copy(asT_hbm.at[0], asrc_tab)
    pltpu.sync_copy(adT_hbm.at[0], adst_tab)
    plsc.subcore_barrier()

    def eb(i, c):
        boff = (wid + NS * NC * i) * BLK
        _edge_block(src_hbm, dst_hbm, h1_hbm, asrc_tab, adst_tab, den_tab,
                    rows, srcb, dstb, gidx, wb, boff, jnp.int32(0))
        pltpu.sync_copy(rows, accum.at[dstb], add=True)
        return c

    lax.fori_loop(0, nblk, eb, 0)
    plsc.subcore_barrier()
    rsl = pl.ds(sid * ROWS_PER_TILE, ROWS_PER_TILE)
    pltpu.sync_copy(accum.at[rsl], num_hbm.at[cid, rsl])
    pltpu.sync_copy(den_tab, den_hbm.at[cid, sid])


def _sc1(src, dst, h1, asT1, adT1):
    mesh = plsc.VectorSubcoreMesh(core_axis_name="c", subcore_axis_name="s",
                                  num_cores=NC, num_subcores=NS)
    f = pl.kernel(
        _sc1_body,
        out_type=[
            jax.ShapeDtypeStruct((NC, NP, HID), jnp.float32),
            jax.ShapeDtypeStruct((NC, NS, NP), jnp.float32),
        ],
        mesh=mesh,
        compiler_params=pltpu.CompilerParams(needs_layout_passes=False),
        scratch_types=[
            pltpu.VMEM_SHARED((NP, HID), jnp.float32),
            pltpu.VMEM((NP,), jnp.float32),
            pltpu.VMEM((NP,), jnp.float32),
            pltpu.VMEM((NP,), jnp.float32),
            pltpu.VMEM((BLK, HID), jnp.float32),
            pltpu.VMEM((BLK,), jnp.int32),
            pltpu.VMEM((BLK,), jnp.int32),
            pltpu.VMEM((BLK,), jnp.int32),
            pltpu.VMEM((BLK,), jnp.float32),
        ],
    )
    return f(src, dst, h1, asT1, adT1)


def _tcfin_body(num_ref, den_ref, b1_ref, out_ref):
    den = jnp.sum(den_ref[...], axis=(0, 1))
    out_ref[...] = ((num_ref[0] + num_ref[1]) / (den[:, None] + 1e-16)
                    + b1_ref[...])


def _tcfin(num1, den1, b1r):
    return pl.pallas_call(
        _tcfin_body,
        grid=(NB,),
        in_specs=[
            pl.BlockSpec((NC, BN, HID), lambda nb: (0, nb, 0)),
            pl.BlockSpec((NC, NS, BN), lambda nb: (0, 0, nb)),
            pl.BlockSpec((1, HID), lambda nb: (0, 0)),
        ],
        out_specs=pl.BlockSpec((BN, HID), lambda nb: (nb, 0)),
        out_shape=jax.ShapeDtypeStruct((NP, HID), jnp.float32),
    )(num1, den1, b1r)


@jax.jit
def kernel(x, edge_index, W0, att_src0, att_dst0, b0, W1, att_src1, att_dst1, b1):
    xp = jnp.pad(x, ((0, NP - N), (0, 0)))
    src = edge_index[0]
    dst = edge_index[1]

    h0T, asT0, adT0 = _tc0(xp, W0, att_src0, att_dst0)
    h0flat = h0T.reshape(HEADS * NP, HID)
    num0, den0 = _sc0(src, dst, h0flat,
                      asT0.reshape(HEADS, NP), adT0.reshape(HEADS, NP))

    b0r = b0.reshape(HEADS, HID)
    W1r = W1.reshape(HEADS, HID, HID)
    h1, asT1, adT1 = _tcmid(num0, den0, b0r, W1r, att_src1, att_dst1)

    num1, den1 = _sc1(src, dst, h1, asT1, adT1)
    outp = _tcfin(num1, den1, b1.reshape(1, HID))
    return outp[:N]
